# baseline (speedup 1.0000x reference)
import functools

import jax
import jax.numpy as jnp
from jax import lax
from jax.experimental import pallas as pl
from jax.experimental.pallas import tpu as pltpu
from jax.experimental.pallas import tpu_sc as plsc

B = 16384
EMB = 32
_NC = 2
_NS = 16
_NW = _NC * _NS
_BPW = B // _NW

_BLK = 2048


def _sc_gather_body(uidx_h, iidx_h, ugt_h, igt_h, umt_h, imt_h,
                    ug_o, ig_o, um_o, im_o,
                    uix_v, iix_v, ug_v, ig_v, um_v, im_v, sem):
    wid = lax.axis_index("s") * _NC + lax.axis_index("c")
    base = wid * _BPW
    pltpu.sync_copy(uidx_h.at[pl.ds(base, _BPW)], uix_v)
    pltpu.sync_copy(iidx_h.at[pl.ds(base, _BPW)], iix_v)
    c1 = pltpu.async_copy(ugt_h.at[uix_v], ug_v, sem)
    c2 = pltpu.async_copy(igt_h.at[iix_v], ig_v, sem)
    c3 = pltpu.async_copy(umt_h.at[uix_v], um_v, sem)
    c4 = pltpu.async_copy(imt_h.at[iix_v], im_v, sem)
    c1.wait()
    c2.wait()
    c3.wait()
    c4.wait()
    pltpu.sync_copy(ug_v, ug_o.at[pl.ds(base, _BPW)])
    pltpu.sync_copy(ig_v, ig_o.at[pl.ds(base, _BPW)])
    pltpu.sync_copy(um_v, um_o.at[pl.ds(base, _BPW)])
    pltpu.sync_copy(im_v, im_o.at[pl.ds(base, _BPW)])


def _sc_gather(user_idx, item_idx, ug_t, ig_t, um_t, im_t):
    mesh = plsc.VectorSubcoreMesh(core_axis_name="c", subcore_axis_name="s")
    row = jax.ShapeDtypeStruct((B, EMB), jnp.float32)
    k = pl.kernel(
        _sc_gather_body,
        out_type=(row, row, row, row),
        mesh=mesh,
        scratch_types=[
            pltpu.VMEM((_BPW,), jnp.int32),
            pltpu.VMEM((_BPW,), jnp.int32),
            pltpu.VMEM((_BPW, EMB), jnp.float32),
            pltpu.VMEM((_BPW, EMB), jnp.float32),
            pltpu.VMEM((_BPW, EMB), jnp.float32),
            pltpu.VMEM((_BPW, EMB), jnp.float32),
            pltpu.SemaphoreType.DMA,
        ],
        compiler_params=pltpu.CompilerParams(use_tc_tiling_on_sc=False),
    )
    return k(user_idx, item_idx, ug_t, ig_t, um_t, im_t)


def _tc_dense_body(ug_r, ig_r, um_r, im_r,
                   w1u_r, w1i_r, b1_r, g1_r, be1_r,
                   w2_r, b2_r, g2_r, be2_r,
                   w3_r, b3_r, g3_r, be3_r,
                   wo_r, bo_r, out_r):
    inv = lax.rsqrt(jnp.float32(1.0 + 1e-5))
    gmf = ug_r[...] * ig_r[...]
    h = (jnp.dot(um_r[...], w1u_r[...], preferred_element_type=jnp.float32)
         + jnp.dot(im_r[...], w1i_r[...], preferred_element_type=jnp.float32)
         + b1_r[...])
    h = jnp.maximum(h * inv * g1_r[...] + be1_r[...], 0.0)
    h = jnp.dot(h, w2_r[...], preferred_element_type=jnp.float32) + b2_r[...]
    h = jnp.maximum(h * inv * g2_r[...] + be2_r[...], 0.0)
    h = jnp.dot(h, w3_r[...], preferred_element_type=jnp.float32) + b3_r[...]
    h = jnp.maximum(h * inv * g3_r[...] + be3_r[...], 0.0)
    wo = wo_r[...]
    logits = (jnp.sum(gmf * wo[0:1, :], axis=1)
              + jnp.sum(h * wo[1:2, :], axis=1)
              + bo_r[0])
    out_r[...] = jax.nn.sigmoid(logits)


def _tc_dense(ug, ig, um, im, W1, b1, g1, be1, W2, b2, g2, be2,
              W3, b3, g3, be3, Wo, bo):
    w1u = W1[:EMB, :]
    w1i = W1[EMB:, :]
    wo2 = Wo[:, 0].reshape(2, EMB)

    bspec = pl.BlockSpec((_BLK, EMB), lambda i: (i, 0))
    wfull = lambda a: pl.BlockSpec(a.shape, lambda i: (0,) * a.ndim)
    grid = B // _BLK
    return pl.pallas_call(
        _tc_dense_body,
        grid=(grid,),
        in_specs=[bspec, bspec, bspec, bspec,
                  wfull(w1u), wfull(w1i), wfull(b1), wfull(g1), wfull(be1),
                  wfull(W2), wfull(b2), wfull(g2), wfull(be2),
                  wfull(W3), wfull(b3), wfull(g3), wfull(be3),
                  wfull(wo2), wfull(bo)],
        out_specs=pl.BlockSpec((_BLK,), lambda i: (i,)),
        out_shape=jax.ShapeDtypeStruct((B,), jnp.float32),
    )(ug, ig, um, im, w1u, w1i, b1, g1, be1, W2, b2, g2, be2,
      W3, b3, g3, be3, wo2, bo)


def kernel(user_idx, item_idx, user_emb_gmf, item_emb_gmf, user_emb_mlp,
           item_emb_mlp, W1, b1, g1, be1, W2, b2, g2, be2, W3, b3, g3, be3,
           Wo, bo):
    ug, ig, um, im = _sc_gather(user_idx, item_idx, user_emb_gmf,
                                item_emb_gmf, user_emb_mlp, item_emb_mlp)
    return _tc_dense(ug, ig, um, im, W1, b1, g1, be1, W2, b2, g2, be2,
                     W3, b3, g3, be3, Wo, bo)
